# Initial kernel scaffold; baseline (speedup 1.0000x reference)
import jax
import jax.numpy as jnp
from jax.experimental import pallas as pl


def kernel(x, edge_index, batch, P_vec, W1, b1, W2, b2, W3, b3, Wo, bo):
    raise NotImplementedError("write your pallas kernel here")


# speedup vs baseline: 7.8268x; 7.8268x over previous
import functools

import jax
import jax.numpy as jnp
from jax import lax
from jax.experimental import pallas as pl
from jax.experimental.pallas import tpu as pltpu
from jax.experimental.pallas import tpu_sc as plsc

N_NODES = 10000
N_EDGES = 320000
D = 128
N_GRAPHS = 64
N_CLASSES = 10

NC, NS = 2, 16
NW = NC * NS
E_PER_W = N_EDGES // NW
EB = 128
N_FULL = E_PER_W // EB
TAIL = E_PER_W - N_FULL * EB
DEG_CHUNK = 2000
AGG_ROWS = 10240
ROWS_PER_SUB = AGG_ROWS // NS

_mesh = plsc.VectorSubcoreMesh(core_axis_name="c", subcore_axis_name="s")


def _wid():
    return lax.axis_index("s") * NC + lax.axis_index("c")


def _sigmoid(v):
    return 1.0 / (1.0 + jnp.exp(-v))


def _deg_body(col_hbm, pv_hbm, out_hbm, degpart, colbuf, pvbuf):
    w = _wid()
    def zero(i, _):
        degpart[pl.ds(i * 16, 16)] = jnp.zeros((16,), jnp.float32)
        return 0
    lax.fori_loop(0, N_NODES // 16, zero, 0)
    base = w * E_PER_W
    def chunk(cidx, _):
        off = base + cidx * DEG_CHUNK
        pltpu.sync_copy(col_hbm.at[pl.ds(off, DEG_CHUNK)], colbuf)
        pltpu.sync_copy(pv_hbm.at[pl.ds(off, DEG_CHUNK)], pvbuf)
        def grp(k, _):
            c16 = colbuf[pl.ds(k * 16, 16)]
            p16 = _sigmoid(pvbuf[pl.ds(k * 16, 16)])
            plsc.addupdate_scatter(degpart, [c16], p16)
            return 0
        lax.fori_loop(0, DEG_CHUNK // 16, grp, 0)
        return 0
    lax.fori_loop(0, E_PER_W // DEG_CHUNK, chunk, 0)
    pltpu.sync_copy(degpart, out_hbm.at[w, 0])


_deg_kernel = pl.kernel(
    _deg_body,
    out_type=jax.ShapeDtypeStruct((NW, 1, N_NODES), jnp.float32),
    mesh=_mesh,
    compiler_params=pltpu.CompilerParams(needs_layout_passes=False),
    scratch_types=[
        pltpu.VMEM((N_NODES,), jnp.float32),
        pltpu.VMEM((DEG_CHUNK,), jnp.int32),
        pltpu.VMEM((DEG_CHUNK,), jnp.float32),
    ],
)


def _dis_body(parts_ref, dis_ref, sl_ref):
    deg = 1.0 + jnp.sum(parts_ref[...], axis=0, keepdims=True)
    dis_ref[...] = lax.rsqrt(deg)
    sl_ref[...] = 1.0 / deg


def _dis_call(parts):
    return pl.pallas_call(
        _dis_body,
        out_shape=[
            jax.ShapeDtypeStruct((1, N_NODES), jnp.float32),
            jax.ShapeDtypeStruct((1, N_NODES), jnp.float32),
        ],
    )(parts)


def _w_body(row_hbm, col_hbm, pv_hbm, dis_hbm, w_hbm,
            disbuf, rowbuf, colbuf, pvbuf, wbuf):
    w = _wid()
    pltpu.sync_copy(dis_hbm, disbuf)
    base = w * E_PER_W
    def chunk(cidx, _):
        off = base + cidx * DEG_CHUNK
        pltpu.sync_copy(row_hbm.at[pl.ds(off, DEG_CHUNK)], rowbuf)
        pltpu.sync_copy(col_hbm.at[pl.ds(off, DEG_CHUNK)], colbuf)
        pltpu.sync_copy(pv_hbm.at[pl.ds(off, DEG_CHUNK)], pvbuf)
        def grp(k, _):
            r16 = rowbuf[pl.ds(k * 16, 16)]
            c16 = colbuf[pl.ds(k * 16, 16)]
            p16 = _sigmoid(pvbuf[pl.ds(k * 16, 16)])
            dr = plsc.load_gather(disbuf, [r16])
            dc = plsc.load_gather(disbuf, [c16])
            wbuf[pl.ds(k * 16, 16)] = dr * p16 * dc
            return 0
        lax.fori_loop(0, DEG_CHUNK // 16, grp, 0)
        pltpu.sync_copy(wbuf, w_hbm.at[pl.ds(off, DEG_CHUNK)])
        return 0
    lax.fori_loop(0, E_PER_W // DEG_CHUNK, chunk, 0)


_w_kernel = pl.kernel(
    _w_body,
    out_type=jax.ShapeDtypeStruct((N_EDGES,), jnp.float32),
    mesh=_mesh,
    compiler_params=pltpu.CompilerParams(needs_layout_passes=False),
    scratch_types=[
        pltpu.VMEM((N_NODES,), jnp.float32),
        pltpu.VMEM((DEG_CHUNK,), jnp.int32),
        pltpu.VMEM((DEG_CHUNK,), jnp.int32),
        pltpu.VMEM((DEG_CHUNK,), jnp.float32),
        pltpu.VMEM((DEG_CHUNK,), jnp.float32),
    ],
)


def _scale_block(gbuf, ubuf, n):
    for k in range(n):
        uv = plsc.load_gather(ubuf, [jnp.full((16,), k, jnp.int32)])
        for j in range(8):
            gbuf[k, j * 16:(j + 1) * 16] = uv * gbuf[k, j * 16:(j + 1) * 16]


def _agg_body(z_hbm, row_hbm, col_hbm, w_hbm, zeros_hbm, out_hbm,
              acc, rowbuf, colbuf, ubuf, gbuf,
              rowbuf_t, colbuf_t, ubuf_t, gbuf_t, sem):
    c = lax.axis_index("c")
    s = lax.axis_index("s")
    w = s * NC + c
    pltpu.sync_copy(zeros_hbm.at[pl.ds(s * ROWS_PER_SUB, ROWS_PER_SUB)],
                    acc.at[pl.ds(s * ROWS_PER_SUB, ROWS_PER_SUB)])
    plsc.subcore_barrier()

    base = w * E_PER_W
    def block(b, _):
        off = base + b * EB
        pltpu.sync_copy(row_hbm.at[pl.ds(off, EB)], rowbuf)
        pltpu.sync_copy(col_hbm.at[pl.ds(off, EB)], colbuf)
        pltpu.sync_copy(w_hbm.at[pl.ds(off, EB)], ubuf)
        pltpu.async_copy(z_hbm.at[rowbuf], gbuf, sem).wait()
        _scale_block(gbuf, ubuf, EB)
        pltpu.sync_copy(gbuf, acc.at[colbuf], add=True)
        return 0
    lax.fori_loop(0, N_FULL, block, 0)

    toff = base + N_FULL * EB
    pltpu.sync_copy(row_hbm.at[pl.ds(toff, TAIL)], rowbuf_t)
    pltpu.sync_copy(col_hbm.at[pl.ds(toff, TAIL)], colbuf_t)
    pltpu.sync_copy(w_hbm.at[pl.ds(toff, TAIL)], ubuf_t)
    pltpu.async_copy(z_hbm.at[rowbuf_t], gbuf_t, sem).wait()
    _scale_block(gbuf_t, ubuf_t, TAIL)
    pltpu.sync_copy(gbuf_t, acc.at[colbuf_t], add=True)

    plsc.subcore_barrier()
    pltpu.sync_copy(acc.at[pl.ds(s * ROWS_PER_SUB, ROWS_PER_SUB)],
                    out_hbm.at[c, pl.ds(s * ROWS_PER_SUB, ROWS_PER_SUB)])


_agg_kernel = pl.kernel(
    _agg_body,
    out_type=jax.ShapeDtypeStruct((NC, AGG_ROWS, D), jnp.float32),
    mesh=_mesh,
    compiler_params=pltpu.CompilerParams(needs_layout_passes=False),
    scratch_types=[
        pltpu.VMEM_SHARED((AGG_ROWS, D), jnp.float32),
        pltpu.VMEM((EB,), jnp.int32),
        pltpu.VMEM((EB,), jnp.int32),
        pltpu.VMEM((EB,), jnp.float32),
        pltpu.VMEM((EB, D), jnp.float32),
        pltpu.VMEM((TAIL,), jnp.int32),
        pltpu.VMEM((TAIL,), jnp.int32),
        pltpu.VMEM((TAIL,), jnp.float32),
        pltpu.VMEM((TAIL, D), jnp.float32),
        pltpu.SemaphoreType.DMA,
    ],
)


RB = 2000


def _mm1_body(x_ref, w_ref, o_ref):
    o_ref[...] = lax.dot_general(
        x_ref[...], w_ref[...], (((1,), (1,)), ((), ())),
        preferred_element_type=jnp.float32)


def _mm1_call(x, W):
    return pl.pallas_call(
        _mm1_body,
        grid=(N_NODES // RB,),
        in_specs=[
            pl.BlockSpec((RB, D), lambda i: (i, 0)),
            pl.BlockSpec((D, D), lambda i: (0, 0)),
        ],
        out_specs=pl.BlockSpec((RB, D), lambda i: (i, 0)),
        out_shape=jax.ShapeDtypeStruct((N_NODES, D), jnp.float32),
    )(x, W)


def _layer_body(p_ref, z_ref, sl_ref, b_ref, w_ref, o_ref, *, relu):
    x = p_ref[0] + p_ref[1] + sl_ref[...] * z_ref[...] + b_ref[...]
    if relu:
        x = jnp.maximum(x, 0.0)
    o_ref[...] = lax.dot_general(
        x, w_ref[...], (((1,), (1,)), ((), ())),
        preferred_element_type=jnp.float32)


def _layer_call(p, z, sl, b, W, relu):
    return pl.pallas_call(
        functools.partial(_layer_body, relu=relu),
        grid=(N_NODES // RB,),
        in_specs=[
            pl.BlockSpec((NC, RB, D), lambda i: (0, i, 0)),
            pl.BlockSpec((RB, D), lambda i: (i, 0)),
            pl.BlockSpec((RB, 1), lambda i: (i, 0)),
            pl.BlockSpec((1, D), lambda i: (0, 0)),
            pl.BlockSpec((D, D), lambda i: (0, 0)),
        ],
        out_specs=pl.BlockSpec((RB, D), lambda i: (i, 0)),
        out_shape=jax.ShapeDtypeStruct((N_NODES, D), jnp.float32),
    )(p, z, sl, b, W)


def _final_body(p_ref, z_ref, sl_ref, b_ref, batch_ref, wo_ref, bo_ref,
                o_ref, acc, cnt):
    i = pl.program_id(0)

    @pl.when(i == 0)
    def _():
        acc[...] = jnp.zeros_like(acc)
        cnt[...] = jnp.zeros_like(cnt)

    h = p_ref[0] + p_ref[1] + sl_ref[...] * z_ref[...] + b_ref[...]
    bt = batch_ref[0]
    gids = lax.broadcasted_iota(jnp.int32, (N_GRAPHS, RB), 0)
    onehot = jnp.where(bt == gids, 1.0, 0.0)
    acc[...] += lax.dot_general(
        onehot, h, (((1,), (0,)), ((), ())), preferred_element_type=jnp.float32)
    cnt[...] += jnp.broadcast_to(
        jnp.sum(onehot, axis=1, keepdims=True), (N_GRAPHS, D))

    @pl.when(i == N_NODES // RB - 1)
    def _():
        pooled = acc[...] / jnp.maximum(cnt[...], 1.0)
        o_ref[...] = lax.dot_general(
            pooled, wo_ref[...], (((1,), (1,)), ((), ())),
            preferred_element_type=jnp.float32) + bo_ref[...]


def _final_call(p, z, sl, b, batch4, Wo, bo):
    return pl.pallas_call(
        _final_body,
        grid=(N_NODES // RB,),
        in_specs=[
            pl.BlockSpec((NC, RB, D), lambda i: (0, i, 0)),
            pl.BlockSpec((RB, D), lambda i: (i, 0)),
            pl.BlockSpec((RB, 1), lambda i: (i, 0)),
            pl.BlockSpec((1, D), lambda i: (0, 0)),
            pl.BlockSpec((1, 1, RB), lambda i: (i, 0, 0)),
            pl.BlockSpec((N_CLASSES, D), lambda i: (0, 0)),
            pl.BlockSpec((1, N_CLASSES), lambda i: (0, 0)),
        ],
        out_specs=pl.BlockSpec((N_GRAPHS, N_CLASSES), lambda i: (0, 0)),
        out_shape=jax.ShapeDtypeStruct((N_GRAPHS, N_CLASSES), jnp.float32),
        scratch_shapes=[
            pltpu.VMEM((N_GRAPHS, D), jnp.float32),
            pltpu.VMEM((N_GRAPHS, D), jnp.float32),
        ],
    )(p, z, sl, b, batch4, Wo, bo)


def kernel(x, edge_index, batch, P_vec, W1, b1, W2, b2, W3, b3, Wo, bo):
    row = edge_index[0]
    col = edge_index[1]
    batch4 = batch.reshape(N_NODES // RB, 1, RB)

    deg_parts = _deg_kernel(col, P_vec).reshape(NW, N_NODES)
    dis, sl = _dis_call(deg_parts)
    dis = dis.reshape(N_NODES)
    sl = sl.reshape(N_NODES, 1)
    w = _w_kernel(row, col, P_vec, dis)

    zeros = jnp.zeros((AGG_ROWS, D), jnp.float32)
    z1 = _mm1_call(x, W1)
    p1 = _agg_kernel(z1, row, col, w, zeros)
    z2 = _layer_call(p1, z1, sl, b1.reshape(1, D), W2, relu=True)
    p2 = _agg_kernel(z2, row, col, w, zeros)
    z3 = _layer_call(p2, z2, sl, b2.reshape(1, D), W3, relu=True)
    p3 = _agg_kernel(z3, row, col, w, zeros)
    return _final_call(p3, z3, sl, b3.reshape(1, D), batch4, Wo,
                       bo.reshape(1, N_CLASSES))


# speedup vs baseline: 26.2362x; 3.3521x over previous
import functools

import jax
import jax.numpy as jnp
from jax import lax
from jax.experimental import pallas as pl
from jax.experimental.pallas import tpu as pltpu
from jax.experimental.pallas import tpu_sc as plsc

N_NODES = 10000
N_EDGES = 320000
D = 128
N_GRAPHS = 64
N_CLASSES = 10

NC, NS = 2, 16
NW = NC * NS
E_PER_W = N_EDGES // NW
EB = 128
N_FULL = E_PER_W // EB
TAIL = E_PER_W - N_FULL * EB
DEG_CHUNK = 2000
AGG_ROWS = 10112
ROWS_PER_SUB = AGG_ROWS // NS

_mesh = plsc.VectorSubcoreMesh(core_axis_name="c", subcore_axis_name="s")


def _wid():
    return lax.axis_index("s") * NC + lax.axis_index("c")


def _sigmoid(v):
    return 1.0 / (1.0 + jnp.exp(-v))


def _deg_body(col_hbm, pv_hbm, out_hbm, degpart, colbuf, pvbuf):
    w = _wid()
    def zero(i, _):
        degpart[pl.ds(i * 16, 16)] = jnp.zeros((16,), jnp.float32)
        return 0
    lax.fori_loop(0, N_NODES // 16, zero, 0)
    base = w * E_PER_W
    def chunk(cidx, _):
        off = base + cidx * DEG_CHUNK
        pltpu.sync_copy(col_hbm.at[pl.ds(off, DEG_CHUNK)], colbuf)
        pltpu.sync_copy(pv_hbm.at[pl.ds(off, DEG_CHUNK)], pvbuf)
        def grp(k, _):
            c16 = colbuf[pl.ds(k * 16, 16)]
            p16 = _sigmoid(pvbuf[pl.ds(k * 16, 16)])
            plsc.addupdate_scatter(degpart, [c16], p16)
            return 0
        lax.fori_loop(0, DEG_CHUNK // 16, grp, 0)
        return 0
    lax.fori_loop(0, E_PER_W // DEG_CHUNK, chunk, 0)
    pltpu.sync_copy(degpart, out_hbm.at[w, 0])


_deg_kernel = pl.kernel(
    _deg_body,
    out_type=jax.ShapeDtypeStruct((NW, 1, N_NODES), jnp.float32),
    mesh=_mesh,
    compiler_params=pltpu.CompilerParams(needs_layout_passes=False),
    scratch_types=[
        pltpu.VMEM((N_NODES,), jnp.float32),
        pltpu.VMEM((DEG_CHUNK,), jnp.int32),
        pltpu.VMEM((DEG_CHUNK,), jnp.float32),
    ],
)


def _dis_body(parts_ref, dis_ref, sl_ref):
    deg = 1.0 + jnp.sum(parts_ref[...], axis=0, keepdims=True)
    dis_ref[...] = lax.rsqrt(deg)
    sl_ref[...] = 1.0 / deg


def _dis_call(parts):
    return pl.pallas_call(
        _dis_body,
        out_shape=[
            jax.ShapeDtypeStruct((1, N_NODES), jnp.float32),
            jax.ShapeDtypeStruct((1, N_NODES), jnp.float32),
        ],
    )(parts)


def _w_body(row_hbm, col_hbm, pv_hbm, dis_hbm, w_hbm,
            disbuf, rowbuf, colbuf, pvbuf, wbuf):
    w = _wid()
    pltpu.sync_copy(dis_hbm, disbuf)
    base = w * E_PER_W
    def chunk(cidx, _):
        off = base + cidx * DEG_CHUNK
        pltpu.sync_copy(row_hbm.at[pl.ds(off, DEG_CHUNK)], rowbuf)
        pltpu.sync_copy(col_hbm.at[pl.ds(off, DEG_CHUNK)], colbuf)
        pltpu.sync_copy(pv_hbm.at[pl.ds(off, DEG_CHUNK)], pvbuf)
        def grp(k, _):
            r16 = rowbuf[pl.ds(k * 16, 16)]
            c16 = colbuf[pl.ds(k * 16, 16)]
            p16 = _sigmoid(pvbuf[pl.ds(k * 16, 16)])
            dr = plsc.load_gather(disbuf, [r16])
            dc = plsc.load_gather(disbuf, [c16])
            wbuf[pl.ds(k * 16, 16)] = dr * p16 * dc
            return 0
        lax.fori_loop(0, DEG_CHUNK // 16, grp, 0)
        pltpu.sync_copy(wbuf, w_hbm.at[pl.ds(off, DEG_CHUNK)])
        return 0
    lax.fori_loop(0, E_PER_W // DEG_CHUNK, chunk, 0)


_w_kernel = pl.kernel(
    _w_body,
    out_type=jax.ShapeDtypeStruct((N_EDGES,), jnp.float32),
    mesh=_mesh,
    compiler_params=pltpu.CompilerParams(needs_layout_passes=False),
    scratch_types=[
        pltpu.VMEM((N_NODES,), jnp.float32),
        pltpu.VMEM((DEG_CHUNK,), jnp.int32),
        pltpu.VMEM((DEG_CHUNK,), jnp.int32),
        pltpu.VMEM((DEG_CHUNK,), jnp.float32),
        pltpu.VMEM((DEG_CHUNK,), jnp.float32),
    ],
)


def _splat(v):
    return jnp.full((16,), v, jnp.int32)


def _scale_rows_fori(buf, wbuf, n):
    def f(k, _):
        uv = plsc.load_gather(wbuf, [_splat(k)])
        for j in range(8):
            buf[k, j * 16:(j + 1) * 16] = uv * buf[k, j * 16:(j + 1) * 16]
        return 0
    lax.fori_loop(0, n, f, 0)


def _agg_body(z_hbm, row_hbm, col_hbm, w_hbm, zeros_hbm, out_hbm,
              acc, row_all, gbuf0, gbuf1, colbuf0, colbuf1, wbuf0, wbuf1,
              colbuf_t, wbuf_t, gbuf_t, stagesem, colsem, wsem, gsem, ssem):
    c = lax.axis_index("c")
    s = lax.axis_index("s")
    wid = s * NC + c
    base = wid * E_PER_W
    gbuf = (gbuf0, gbuf1)
    colbuf = (colbuf0, colbuf1)
    wbuf = (wbuf0, wbuf1)

    pltpu.async_copy(zeros_hbm.at[pl.ds(s * ROWS_PER_SUB, ROWS_PER_SUB)],
                     acc.at[pl.ds(s * ROWS_PER_SUB, ROWS_PER_SUB)], stagesem)
    pltpu.sync_copy(row_hbm.at[pl.ds(base, E_PER_W)], row_all)
    pltpu.async_copy(col_hbm.at[pl.ds(base, EB)], colbuf0, colsem)
    pltpu.async_copy(w_hbm.at[pl.ds(base, EB)], wbuf0, wsem)
    pltpu.async_copy(z_hbm.at[row_all.at[pl.ds(0, EB)]], gbuf0, gsem)
    pltpu.make_async_copy(zeros_hbm.at[pl.ds(0, ROWS_PER_SUB)],
                          acc.at[pl.ds(0, ROWS_PER_SUB)], stagesem).wait()
    plsc.subcore_barrier()

    def outer(g, _):
        for b in range(2):
            i = g * 2 + b
            pltpu.make_async_copy(z_hbm.at[pl.ds(0, EB)], gbuf[b],
                                  gsem).wait()

            @pl.when(i >= 1)
            def _():
                pltpu.make_async_copy(z_hbm.at[pl.ds(0, EB)], gbuf[1 - b],
                                      ssem).wait()

            @pl.when(i + 1 < N_FULL)
            def _():
                off = base + (i + 1) * EB
                pltpu.async_copy(
                    z_hbm.at[row_all.at[pl.ds((i + 1) * EB, EB)]],
                    gbuf[1 - b], gsem)
                pltpu.async_copy(col_hbm.at[pl.ds(off, EB)], colbuf[1 - b],
                                 colsem)
                pltpu.async_copy(w_hbm.at[pl.ds(off, EB)], wbuf[1 - b], wsem)

            pltpu.make_async_copy(w_hbm.at[pl.ds(base, EB)], wbuf[b],
                                  wsem).wait()
            _scale_rows_fori(gbuf[b], wbuf[b], EB)
            pltpu.make_async_copy(col_hbm.at[pl.ds(base, EB)], colbuf[b],
                                  colsem).wait()
            pltpu.async_copy(gbuf[b], acc.at[colbuf[b]], ssem, add=True)
        return 0
    lax.fori_loop(0, N_FULL // 2, outer, 0)
    pltpu.make_async_copy(z_hbm.at[pl.ds(0, EB)], gbuf1, ssem).wait()

    toff = base + N_FULL * EB
    pltpu.async_copy(col_hbm.at[pl.ds(toff, TAIL)], colbuf_t, colsem)
    pltpu.async_copy(w_hbm.at[pl.ds(toff, TAIL)], wbuf_t, wsem)
    pltpu.async_copy(z_hbm.at[row_all.at[pl.ds(N_FULL * EB, TAIL)]],
                     gbuf_t, gsem).wait()
    pltpu.make_async_copy(w_hbm.at[pl.ds(base, TAIL)], wbuf_t, wsem).wait()
    _scale_rows_fori(gbuf_t, wbuf_t, TAIL)
    pltpu.make_async_copy(col_hbm.at[pl.ds(base, TAIL)], colbuf_t,
                          colsem).wait()
    pltpu.sync_copy(gbuf_t, acc.at[colbuf_t], add=True)

    plsc.subcore_barrier()
    pltpu.sync_copy(acc.at[pl.ds(s * ROWS_PER_SUB, ROWS_PER_SUB)],
                    out_hbm.at[c, pl.ds(s * ROWS_PER_SUB, ROWS_PER_SUB)])


_agg_kernel = pl.kernel(
    _agg_body,
    out_type=jax.ShapeDtypeStruct((NC, AGG_ROWS, D), jnp.float32),
    mesh=_mesh,
    compiler_params=pltpu.CompilerParams(needs_layout_passes=False),
    scratch_types=[
        pltpu.VMEM_SHARED((AGG_ROWS, D), jnp.float32),
        pltpu.VMEM((E_PER_W,), jnp.int32),
        pltpu.VMEM((EB, D), jnp.float32),
        pltpu.VMEM((EB, D), jnp.float32),
        pltpu.VMEM((EB,), jnp.int32),
        pltpu.VMEM((EB,), jnp.int32),
        pltpu.VMEM((EB,), jnp.float32),
        pltpu.VMEM((EB,), jnp.float32),
        pltpu.VMEM((TAIL,), jnp.int32),
        pltpu.VMEM((TAIL,), jnp.float32),
        pltpu.VMEM((TAIL, D), jnp.float32),
        pltpu.SemaphoreType.DMA,
        pltpu.SemaphoreType.DMA,
        pltpu.SemaphoreType.DMA,
        pltpu.SemaphoreType.DMA,
        pltpu.SemaphoreType.DMA,
    ],
)


RB = 2000


def _mm1_body(x_ref, w_ref, o_ref):
    o_ref[...] = lax.dot_general(
        x_ref[...], w_ref[...], (((1,), (1,)), ((), ())),
        preferred_element_type=jnp.float32)


def _mm1_call(x, W):
    return pl.pallas_call(
        _mm1_body,
        grid=(N_NODES // RB,),
        in_specs=[
            pl.BlockSpec((RB, D), lambda i: (i, 0)),
            pl.BlockSpec((D, D), lambda i: (0, 0)),
        ],
        out_specs=pl.BlockSpec((RB, D), lambda i: (i, 0)),
        out_shape=jax.ShapeDtypeStruct((N_NODES, D), jnp.float32),
    )(x, W)


def _layer_body(p_ref, z_ref, sl_ref, b_ref, w_ref, o_ref, *, relu):
    x = p_ref[0] + p_ref[1] + sl_ref[...] * z_ref[...] + b_ref[...]
    if relu:
        x = jnp.maximum(x, 0.0)
    o_ref[...] = lax.dot_general(
        x, w_ref[...], (((1,), (1,)), ((), ())),
        preferred_element_type=jnp.float32)


def _layer_call(p, z, sl, b, W, relu):
    return pl.pallas_call(
        functools.partial(_layer_body, relu=relu),
        grid=(N_NODES // RB,),
        in_specs=[
            pl.BlockSpec((NC, RB, D), lambda i: (0, i, 0)),
            pl.BlockSpec((RB, D), lambda i: (i, 0)),
            pl.BlockSpec((RB, 1), lambda i: (i, 0)),
            pl.BlockSpec((1, D), lambda i: (0, 0)),
            pl.BlockSpec((D, D), lambda i: (0, 0)),
        ],
        out_specs=pl.BlockSpec((RB, D), lambda i: (i, 0)),
        out_shape=jax.ShapeDtypeStruct((N_NODES, D), jnp.float32),
    )(p, z, sl, b, W)


def _final_body(p_ref, z_ref, sl_ref, b_ref, batch_ref, wo_ref, bo_ref,
                o_ref, acc, cnt):
    i = pl.program_id(0)

    @pl.when(i == 0)
    def _():
        acc[...] = jnp.zeros_like(acc)
        cnt[...] = jnp.zeros_like(cnt)

    h = p_ref[0] + p_ref[1] + sl_ref[...] * z_ref[...] + b_ref[...]
    bt = batch_ref[0]
    gids = lax.broadcasted_iota(jnp.int32, (N_GRAPHS, RB), 0)
    onehot = jnp.where(bt == gids, 1.0, 0.0)
    acc[...] += lax.dot_general(
        onehot, h, (((1,), (0,)), ((), ())), preferred_element_type=jnp.float32)
    cnt[...] += jnp.broadcast_to(
        jnp.sum(onehot, axis=1, keepdims=True), (N_GRAPHS, D))

    @pl.when(i == N_NODES // RB - 1)
    def _():
        pooled = acc[...] / jnp.maximum(cnt[...], 1.0)
        o_ref[...] = lax.dot_general(
            pooled, wo_ref[...], (((1,), (1,)), ((), ())),
            preferred_element_type=jnp.float32) + bo_ref[...]


def _final_call(p, z, sl, b, batch4, Wo, bo):
    return pl.pallas_call(
        _final_body,
        grid=(N_NODES // RB,),
        in_specs=[
            pl.BlockSpec((NC, RB, D), lambda i: (0, i, 0)),
            pl.BlockSpec((RB, D), lambda i: (i, 0)),
            pl.BlockSpec((RB, 1), lambda i: (i, 0)),
            pl.BlockSpec((1, D), lambda i: (0, 0)),
            pl.BlockSpec((1, 1, RB), lambda i: (i, 0, 0)),
            pl.BlockSpec((N_CLASSES, D), lambda i: (0, 0)),
            pl.BlockSpec((1, N_CLASSES), lambda i: (0, 0)),
        ],
        out_specs=pl.BlockSpec((N_GRAPHS, N_CLASSES), lambda i: (0, 0)),
        out_shape=jax.ShapeDtypeStruct((N_GRAPHS, N_CLASSES), jnp.float32),
        scratch_shapes=[
            pltpu.VMEM((N_GRAPHS, D), jnp.float32),
            pltpu.VMEM((N_GRAPHS, D), jnp.float32),
        ],
    )(p, z, sl, b, batch4, Wo, bo)


def kernel(x, edge_index, batch, P_vec, W1, b1, W2, b2, W3, b3, Wo, bo):
    row = edge_index[0]
    col = edge_index[1]
    batch4 = batch.reshape(N_NODES // RB, 1, RB)

    deg_parts = _deg_kernel(col, P_vec).reshape(NW, N_NODES)
    dis, sl = _dis_call(deg_parts)
    dis = dis.reshape(N_NODES)
    sl = sl.reshape(N_NODES, 1)
    w = _w_kernel(row, col, P_vec, dis)

    zeros = jnp.zeros((AGG_ROWS, D), jnp.float32)
    z1 = _mm1_call(x, W1)
    p1 = _agg_kernel(z1, row, col, w, zeros)
    z2 = _layer_call(p1, z1, sl, b1.reshape(1, D), W2, relu=True)
    p2 = _agg_kernel(z2, row, col, w, zeros)
    z3 = _layer_call(p2, z2, sl, b2.reshape(1, D), W3, relu=True)
    p3 = _agg_kernel(z3, row, col, w, zeros)
    return _final_call(p3, z3, sl, b3.reshape(1, D), batch4, Wo,
                       bo.reshape(1, N_CLASSES))
